# Initial kernel scaffold; baseline (speedup 1.0000x reference)
import jax
import jax.numpy as jnp
from jax.experimental import pallas as pl


def kernel(x, edge_index, W1, b1, W2, b2):
    raise NotImplementedError("write your pallas kernel here")


# speedup vs baseline: 14.5315x; 14.5315x over previous
import functools

import jax
import jax.numpy as jnp
from jax import lax
from jax.experimental import pallas as pl
from jax.experimental.pallas import tpu as pltpu
from jax.experimental.pallas import tpu_sc as plsc

N = 10000
E = 320000
D_IN = 128
D_HID = 128
D_OUT = 7
D_PAD = 16

NC, NS = 2, 16
NW = NC * NS
EPT = E // NW
C = 80
NCHUNK = EPT // C
N_PAD = 10240
RPT = N_PAD // NS
ZROWS = 128
NZCP = RPT // ZROWS


def _make_prop(D, gather):
    mesh = plsc.VectorSubcoreMesh(core_axis_name="c", subcore_axis_name="s")
    scratch = [
        pltpu.VMEM((C,), jnp.int32),
        pltpu.VMEM((C,), jnp.int32),
        pltpu.VMEM((C, D), jnp.float32),
        pltpu.VMEM((ZROWS, D), jnp.float32),
        pltpu.VMEM_SHARED((N_PAD, D), jnp.float32),
        pltpu.SemaphoreType.DMA,
    ]

    def body(*refs):
        if gather:
            table, srcv, dstv, out, sidx, didx, rows, zbuf, acc, sem = refs
        else:
            srcv, dstv, out, sidx, didx, rows, zbuf, acc, sem = refs
            table = None
        cid = lax.axis_index("c")
        sid = lax.axis_index("s")
        wid = cid * NS + sid

        zero16 = jnp.zeros((16,), jnp.float32)

        def zrow(i, carry):
            for j in range(D // 16):
                zbuf[i, pl.ds(j * 16, 16)] = zero16
            return carry

        lax.fori_loop(0, ZROWS, zrow, 0)

        def zcp(j, carry):
            pltpu.sync_copy(zbuf, acc.at[pl.ds(sid * RPT + j * ZROWS, ZROWS)])
            return carry

        lax.fori_loop(0, NZCP, zcp, 0)

        if not gather:
            one16 = jnp.ones((16,), jnp.float32)

            def orow(i, carry):
                for j in range(D // 16):
                    rows[i, pl.ds(j * 16, 16)] = one16
                return carry

            lax.fori_loop(0, C, orow, 0)

        plsc.subcore_barrier()

        ebase = wid * EPT

        def step(g, carry):
            off = ebase + g * C
            pltpu.sync_copy(dstv.at[pl.ds(off, C)], didx)
            if gather:
                pltpu.sync_copy(srcv.at[pl.ds(off, C)], sidx)
                pltpu.async_copy(table.at[sidx], rows, sem).wait()
            pltpu.sync_copy(rows, acc.at[didx], add=True)
            return carry

        lax.fori_loop(0, NCHUNK, step, 0)

        plsc.subcore_barrier()

        def wout(j, carry):
            r0 = sid * RPT + j * ZROWS
            pltpu.sync_copy(acc.at[pl.ds(r0, ZROWS)],
                            out.at[cid, pl.ds(r0, ZROWS)])
            return carry

        lax.fori_loop(0, NZCP, wout, 0)

    return pl.kernel(
        body,
        out_type=jax.ShapeDtypeStruct((NC, N_PAD, D), jnp.float32),
        mesh=mesh,
        scratch_types=scratch,
        compiler_params=pltpu.CompilerParams(use_tc_tiling_on_sc=False),
    )


_prop128 = _make_prop(D_HID, gather=True)
_prop16 = _make_prop(D_PAD, gather=True)
_degree = _make_prop(D_PAD, gather=False)


def _dinv(degp_ref):
    deg = degp_ref[0, :N, :1] + degp_ref[1, :N, :1] + 1.0
    return lax.rsqrt(deg)


def _tc1_body(x_ref, w_ref, degp_ref, xs_ref):
    dinv = _dinv(degp_ref)
    xw = jnp.dot(x_ref[...], w_ref[...], preferred_element_type=jnp.float32)
    xs_ref[...] = xw * dinv


def _tc2_body(acc_ref, xs_ref, degp_ref, w2_ref, b1_ref, zs_ref):
    dinv = _dinv(degp_ref)
    h = (acc_ref[0, :N] + acc_ref[1, :N] + xs_ref[...]) * dinv + b1_ref[...]
    h = jnp.maximum(h, 0.0)
    z = jnp.dot(h, w2_ref[...], preferred_element_type=jnp.float32)
    zs_ref[...] = z * dinv


def _tc3_body(acc_ref, zs_ref, degp_ref, b2_ref, out_ref):
    dinv = _dinv(degp_ref)
    out_ref[...] = (acc_ref[0, :N] + acc_ref[1, :N] + zs_ref[...]) * dinv \
        + b2_ref[...]


_tc1 = pl.pallas_call(
    _tc1_body, out_shape=jax.ShapeDtypeStruct((N, D_HID), jnp.float32))
_tc2 = pl.pallas_call(
    _tc2_body, out_shape=jax.ShapeDtypeStruct((N, D_PAD), jnp.float32))
_tc3 = pl.pallas_call(
    _tc3_body, out_shape=jax.ShapeDtypeStruct((N, D_PAD), jnp.float32))


def kernel(x, edge_index, W1, b1, W2, b2):
    src = edge_index[0]
    dst = edge_index[1]
    w2p = jnp.zeros((D_HID, D_PAD), jnp.float32).at[:, :D_OUT].set(W2)
    b1r = b1.reshape(1, D_HID)
    b2r = jnp.zeros((1, D_PAD), jnp.float32).at[0, :D_OUT].set(b2)

    degp = _degree(src, dst)
    xs = _tc1(x, W1, degp)
    acc1 = _prop128(xs, src, dst)
    zs = _tc2(acc1, xs, degp, w2p, b1r)
    acc2 = _prop16(zs, src, dst)
    out16 = _tc3(acc2, zs, degp, b2r)
    return out16[:, :D_OUT]


# speedup vs baseline: 30.8827x; 2.1252x over previous
import functools

import jax
import jax.numpy as jnp
from jax import lax
from jax.experimental import pallas as pl
from jax.experimental.pallas import tpu as pltpu
from jax.experimental.pallas import tpu_sc as plsc

N = 10000
E = 320000
D_IN = 128
D_HID = 128
D_OUT = 7
D_PAD = 16

NC, NS = 2, 16
NW = NC * NS
EPT = E // NW
C = 80
NCHUNK = EPT // C
N_PAD = 10240
RPT = N_PAD // NS
ZROWS = C
NZCP = RPT // ZROWS


def _make_prop(D, gather):
    mesh = plsc.VectorSubcoreMesh(core_axis_name="c", subcore_axis_name="s")
    scratch = [
        pltpu.VMEM((NCHUNK, C), jnp.int32),
        pltpu.VMEM((NCHUNK, C), jnp.int32),
        pltpu.VMEM((C, D), jnp.float32),
        pltpu.VMEM((C, D), jnp.float32),
        pltpu.VMEM_SHARED((N_PAD, D), jnp.float32),
        pltpu.SemaphoreType.DMA,
        pltpu.SemaphoreType.DMA,
        pltpu.SemaphoreType.DMA,
        pltpu.SemaphoreType.DMA,
    ]

    def body(*refs):
        if gather:
            (table, srcv, dstv, out, sidx, didx, rows0, rows1, acc,
             gsem0, gsem1, ssem0, ssem1) = refs
        else:
            (srcv, dstv, out, sidx, didx, rows0, rows1, acc,
             gsem0, gsem1, ssem0, ssem1) = refs
            table = None
        rows = (rows0, rows1)
        gsem = (gsem0, gsem1)
        ssem = (ssem0, ssem1)
        cid = lax.axis_index("c")
        sid = lax.axis_index("s")
        wid = cid * NS + sid

        zero16 = jnp.zeros((16,), jnp.float32)

        def zrow(i, carry):
            for j in range(D // 16):
                rows0[i, pl.ds(j * 16, 16)] = zero16
            return carry

        lax.fori_loop(0, ZROWS, zrow, 0)

        def zcp(j, carry):
            pltpu.sync_copy(rows0, acc.at[pl.ds(sid * RPT + j * ZROWS, ZROWS)])
            return carry

        lax.fori_loop(0, NZCP, zcp, 0)

        pltpu.sync_copy(dstv.at[wid], didx)
        if gather:
            pltpu.sync_copy(srcv.at[wid], sidx)
        else:
            one16 = jnp.ones((16,), jnp.float32)

            def orow(i, carry):
                for j in range(D // 16):
                    rows0[i, pl.ds(j * 16, 16)] = one16
                return carry

            lax.fori_loop(0, C, orow, 0)

        plsc.subcore_barrier()

        if gather:
            def gfire(g, b):
                pltpu.async_copy(table.at[sidx.at[g]], rows[b], gsem[b])

            def gwait(g, b):
                pltpu.make_async_copy(
                    table.at[sidx.at[g]], rows[b], gsem[b]).wait()

            def sfire(g, b):
                pltpu.async_copy(rows[b], acc.at[didx.at[g]], ssem[b],
                                 add=True)

            def swait(g, b):
                pltpu.make_async_copy(
                    rows[b], acc.at[didx.at[g]], ssem[b]).wait()

            gfire(0, 0)
            gfire(1, 1)
            T = NCHUNK // 2

            def step(t, carry):
                g = 2 * t
                gwait(g, 0)
                sfire(g, 0)
                gwait(g + 1, 1)
                sfire(g + 1, 1)

                @pl.when(t < T - 1)
                def _():
                    swait(g, 0)
                    gfire(g + 2, 0)
                    swait(g + 1, 1)
                    gfire(g + 3, 1)

                return carry

            lax.fori_loop(0, T, step, 0)
            g_last = NCHUNK - 1
            swait(2 * T - 2, 0)
            gfire(g_last, 0)
            gwait(g_last, 0)
            sfire(g_last, 0)
            swait(2 * T - 1, 1)
            swait(g_last, 0)
        else:
            def dfire(g):
                pltpu.async_copy(rows0, acc.at[didx.at[g]], ssem0, add=True)

            def dwait(g):
                pltpu.make_async_copy(rows0, acc.at[didx.at[g]], ssem0).wait()

            dfire(0)

            def step(t, carry):
                dfire(t)
                dwait(t - 1)
                return carry

            lax.fori_loop(1, NCHUNK, step, 0)
            dwait(NCHUNK - 1)

        plsc.subcore_barrier()

        def wout(j, carry):
            r0 = sid * RPT + j * ZROWS
            pltpu.sync_copy(acc.at[pl.ds(r0, ZROWS)],
                            out.at[cid, pl.ds(r0, ZROWS)])
            return carry

        lax.fori_loop(0, NZCP, wout, 0)

    return pl.kernel(
        body,
        out_type=jax.ShapeDtypeStruct((NC, N_PAD, D), jnp.float32),
        mesh=mesh,
        scratch_types=scratch,
        compiler_params=pltpu.CompilerParams(use_tc_tiling_on_sc=False),
    )


_prop128 = _make_prop(D_HID, gather=True)
_prop16 = _make_prop(D_PAD, gather=True)
_degree = _make_prop(D_PAD, gather=False)


def _dinv(degp_ref):
    deg = degp_ref[0, :N, :1] + degp_ref[1, :N, :1] + 1.0
    return lax.rsqrt(deg)


def _tc1_body(x_ref, w_ref, degp_ref, xs_ref):
    dinv = _dinv(degp_ref)
    xw = jnp.dot(x_ref[...], w_ref[...], preferred_element_type=jnp.float32)
    xs_ref[...] = xw * dinv


def _tc2_body(acc_ref, xs_ref, degp_ref, w2_ref, b1_ref, zs_ref):
    dinv = _dinv(degp_ref)
    h = (acc_ref[0, :N] + acc_ref[1, :N] + xs_ref[...]) * dinv + b1_ref[...]
    h = jnp.maximum(h, 0.0)
    z = jnp.dot(h, w2_ref[...], preferred_element_type=jnp.float32)
    zs_ref[...] = z * dinv


def _tc3_body(acc_ref, zs_ref, degp_ref, b2_ref, out_ref):
    dinv = _dinv(degp_ref)
    out_ref[...] = (acc_ref[0, :N] + acc_ref[1, :N] + zs_ref[...]) * dinv \
        + b2_ref[...]


_tc1 = pl.pallas_call(
    _tc1_body, out_shape=jax.ShapeDtypeStruct((N, D_HID), jnp.float32))
_tc2 = pl.pallas_call(
    _tc2_body, out_shape=jax.ShapeDtypeStruct((N, D_PAD), jnp.float32))
_tc3 = pl.pallas_call(
    _tc3_body, out_shape=jax.ShapeDtypeStruct((N, D_PAD), jnp.float32))


def kernel(x, edge_index, W1, b1, W2, b2):
    src = edge_index[0].reshape(NW, NCHUNK, C)
    dst = edge_index[1].reshape(NW, NCHUNK, C)
    w2p = jnp.zeros((D_HID, D_PAD), jnp.float32).at[:, :D_OUT].set(W2)
    b1r = b1.reshape(1, D_HID)
    b2r = jnp.zeros((1, D_PAD), jnp.float32).at[0, :D_OUT].set(b2)

    degp = _degree(src, dst)
    xs = _tc1(x, W1, degp)
    acc1 = _prop128(xs, src, dst)
    zs = _tc2(acc1, xs, degp, w2p, b1r)
    acc2 = _prop16(zs, src, dst)
    out16 = _tc3(acc2, zs, degp, b2r)
    return out16[:, :D_OUT]
